# baseline (speedup 1.0000x reference)
import jax
import jax.numpy as jnp
from jax import lax
from jax.experimental import pallas as pl
from jax.experimental.pallas import tpu as pltpu
from jax.experimental.pallas import tpu_sc as plsc

N = 10000
NP = 10112
E = 320000
D = 128

NC = 2
NS = 16
NW = NC * NS
CHUNK = 128
ROWS_TILE = 80
EROWS = NW * ROWS_TILE
PHROWS = 40
NPH = ROWS_TILE // PHROWS
NPAIR = PHROWS // 2
RPT = NP // NS
NZF = RPT // CHUNK
RZ = RPT - NZF * CHUNK


def _sc_body(x_hbm, src_hbm, dst_hbm, psum_hbm, pcnt_hbm,
             acc, idx_s, idx_d, rows_v, rows_b, sem, sem2):
  c = lax.axis_index("c")
  s = lax.axis_index("s")
  wid = s * NC + c

  zeros16 = jnp.zeros((16,), jnp.float32)
  ones16 = jnp.ones((16,), jnp.float32)

  def fillz(i, carry):
    for j in range(D // 16):
      rows_v[i, pl.ds(j * 16, 16)] = zeros16
    return carry
  lax.fori_loop(0, CHUNK, fillz, 0)

  base = s * RPT
  for b in range(NZF):
    pltpu.sync_copy(rows_v, acc.at[pl.ds(base + b * CHUNK, CHUNK)])
  pltpu.sync_copy(rows_v.at[pl.ds(0, RZ)], acc.at[pl.ds(base + NZF * CHUNK, RZ)])

  plsc.subcore_barrier()

  row0 = wid * ROWS_TILE
  for ph in range(NPH):
    pltpu.sync_copy(src_hbm.at[pl.ds(row0 + ph * PHROWS, PHROWS)], idx_s)
    pltpu.sync_copy(dst_hbm.at[pl.ds(row0 + ph * PHROWS, PHROWS)], idx_d)

    def edge_pair(j, carry):
      ga = pltpu.async_copy(x_hbm.at[idx_s.at[2 * j]], rows_v, sem)
      gb = pltpu.async_copy(x_hbm.at[idx_s.at[2 * j + 1]], rows_b, sem2)
      ga.wait()
      pltpu.sync_copy(rows_v, acc.at[idx_d.at[2 * j]], add=True)
      gb.wait()
      pltpu.sync_copy(rows_b, acc.at[idx_d.at[2 * j + 1]], add=True)
      return carry
    lax.fori_loop(0, NPAIR, edge_pair, 0)

  plsc.subcore_barrier()

  hb = c * NP + base
  for b in range(NZF):
    pltpu.sync_copy(acc.at[pl.ds(base + b * CHUNK, CHUNK)], rows_v)
    pltpu.sync_copy(rows_v, psum_hbm.at[pl.ds(hb + b * CHUNK, CHUNK)])
  pltpu.sync_copy(acc.at[pl.ds(base + NZF * CHUNK, RZ)], rows_v.at[pl.ds(0, RZ)])
  pltpu.sync_copy(rows_v.at[pl.ds(0, RZ)], psum_hbm.at[pl.ds(hb + NZF * CHUNK, RZ)])

  lax.fori_loop(0, CHUNK, fillz, 0)
  for b in range(NZF):
    pltpu.sync_copy(rows_v, acc.at[pl.ds(base + b * CHUNK, CHUNK)])
  pltpu.sync_copy(rows_v.at[pl.ds(0, RZ)], acc.at[pl.ds(base + NZF * CHUNK, RZ)])

  plsc.subcore_barrier()

  def fillo(i, carry):
    for j in range(D // 16):
      rows_v[i, pl.ds(j * 16, 16)] = ones16
      rows_b[i, pl.ds(j * 16, 16)] = ones16
    return carry
  lax.fori_loop(0, CHUNK, fillo, 0)

  for ph in range(NPH):
    pltpu.sync_copy(dst_hbm.at[pl.ds(row0 + ph * PHROWS, PHROWS)], idx_d)

    def cnt_pair(j, carry):
      s0 = pltpu.async_copy(rows_v, acc.at[idx_d.at[2 * j]], sem, add=True)
      s1 = pltpu.async_copy(rows_b, acc.at[idx_d.at[2 * j + 1]], sem2, add=True)
      s0.wait()
      s1.wait()
      return carry
    lax.fori_loop(0, NPAIR, cnt_pair, 0)

  plsc.subcore_barrier()

  for b in range(NZF):
    pltpu.sync_copy(acc.at[pl.ds(base + b * CHUNK, CHUNK)], rows_v)
    pltpu.sync_copy(rows_v, pcnt_hbm.at[pl.ds(hb + b * CHUNK, CHUNK)])
  pltpu.sync_copy(acc.at[pl.ds(base + NZF * CHUNK, RZ)], rows_v.at[pl.ds(0, RZ)])
  pltpu.sync_copy(rows_v.at[pl.ds(0, RZ)], pcnt_hbm.at[pl.ds(hb + NZF * CHUNK, RZ)])


@jax.jit
def _sc_aggregate(x, src2d, dst2d):
  mesh = plsc.VectorSubcoreMesh(core_axis_name="c", subcore_axis_name="s")
  return pl.kernel(
      _sc_body,
      mesh=mesh,
      out_type=(
          jax.ShapeDtypeStruct((NC * NP, D), jnp.float32),
          jax.ShapeDtypeStruct((NC * NP, D), jnp.float32),
      ),
      scratch_types=[
          pltpu.VMEM_SHARED((NP, D), jnp.float32),
          pltpu.VMEM((PHROWS, CHUNK), jnp.int32),
          pltpu.VMEM((PHROWS, CHUNK), jnp.int32),
          pltpu.VMEM((CHUNK, D), jnp.float32),
          pltpu.VMEM((CHUNK, D), jnp.float32),
          pltpu.SemaphoreType.DMA,
          pltpu.SemaphoreType.DMA,
      ],
  )(x, src2d, dst2d)


def _combine_body(ps_ref, pc_ref, out_ref):
  ssum = ps_ref[0:N, :] + ps_ref[NP:NP + N, :]
  cn = pc_ref[0:N, 0:1] + pc_ref[NP:NP + N, 0:1]
  out_ref[...] = ssum / jnp.maximum(cn, 1.0)


@jax.jit
def _tc_combine(psum, pcnt):
  return pl.pallas_call(
      _combine_body,
      out_shape=jax.ShapeDtypeStruct((N, D), jnp.float32),
  )(psum, pcnt)


def kernel(x, edge_index):
  src = edge_index[0].astype(jnp.int32)
  dst = edge_index[1].astype(jnp.int32)
  pad = EROWS * CHUNK - E
  src2d = jnp.concatenate([src, jnp.zeros((pad,), jnp.int32)]).reshape(EROWS, CHUNK)
  dst2d = jnp.concatenate([dst, jnp.full((pad,), NP - 1, jnp.int32)]).reshape(EROWS, CHUNK)
  psum, pcnt = _sc_aggregate(x, src2d, dst2d)
  return _tc_combine(psum, pcnt)


# speedup vs baseline: 1.0401x; 1.0401x over previous
import jax
import jax.numpy as jnp
from jax import lax
from jax.experimental import pallas as pl
from jax.experimental.pallas import tpu as pltpu
from jax.experimental.pallas import tpu_sc as plsc

N = 10000
NP = 10112
E = 320000
D = 128

NC = 2
NS = 16
NW = NC * NS
CHUNK = 128
ROWS_TILE = 80
EROWS = NW * ROWS_TILE
PHROWS = 16
NPH = ROWS_TILE // PHROWS
RPT = NP // NS
NZF = RPT // CHUNK
RZ = RPT - NZF * CHUNK


def _sc_body(x_hbm, src_hbm, dst_hbm, psum_hbm, pcnt_hbm,
             acc, idx_s, idx_d, rows_v, rows_b, sem, sem2, sem3, sem4):
  ssems = (sem3, sem4)
  c = lax.axis_index("c")
  s = lax.axis_index("s")
  wid = s * NC + c

  zeros16 = jnp.zeros((16,), jnp.float32)
  ones16 = jnp.ones((16,), jnp.float32)

  def fillz(i, carry):
    for j in range(D // 16):
      rows_v[i, pl.ds(j * 16, 16)] = zeros16
    return carry
  lax.fori_loop(0, CHUNK, fillz, 0)

  base = s * RPT
  for b in range(NZF):
    pltpu.sync_copy(rows_v, acc.at[pl.ds(base + b * CHUNK, CHUNK)])
  pltpu.sync_copy(rows_v.at[pl.ds(0, RZ)], acc.at[pl.ds(base + NZF * CHUNK, RZ)])

  plsc.subcore_barrier()

  row0 = wid * ROWS_TILE
  rows = (rows_v, rows_b)
  gsems = (sem, sem2)

  def gather_phase(ph, carry):
    pltpu.sync_copy(src_hbm.at[pl.ds(row0 + ph * PHROWS, PHROWS)], idx_s)
    pltpu.sync_copy(dst_hbm.at[pl.ds(row0 + ph * PHROWS, PHROWS)], idx_d)
    g = pltpu.async_copy(x_hbm.at[idx_s.at[0]], rows[0], gsems[0])
    handles = {0: g}
    for k in range(PHROWS):
      b = k % 2
      handles[k].wait()
      if k + 1 < PHROWS:
        handles[k + 1] = pltpu.async_copy(
            x_hbm.at[idx_s.at[k + 1]], rows[1 - b], gsems[1 - b])
      sf = pltpu.async_copy(rows[b], acc.at[idx_d.at[k]], ssems[b], add=True)
      sf.wait()
    return carry
  lax.fori_loop(0, NPH, gather_phase, 0)

  plsc.subcore_barrier()

  hb = c * NP + base
  for b in range(NZF):
    pltpu.sync_copy(acc.at[pl.ds(base + b * CHUNK, CHUNK)], rows_v)
    pltpu.sync_copy(rows_v, psum_hbm.at[pl.ds(hb + b * CHUNK, CHUNK)])
  pltpu.sync_copy(acc.at[pl.ds(base + NZF * CHUNK, RZ)], rows_v.at[pl.ds(0, RZ)])
  pltpu.sync_copy(rows_v.at[pl.ds(0, RZ)], psum_hbm.at[pl.ds(hb + NZF * CHUNK, RZ)])

  lax.fori_loop(0, CHUNK, fillz, 0)
  for b in range(NZF):
    pltpu.sync_copy(rows_v, acc.at[pl.ds(base + b * CHUNK, CHUNK)])
  pltpu.sync_copy(rows_v.at[pl.ds(0, RZ)], acc.at[pl.ds(base + NZF * CHUNK, RZ)])

  plsc.subcore_barrier()

  def fillo(i, carry):
    for j in range(D // 16):
      rows_v[i, pl.ds(j * 16, 16)] = ones16
      rows_b[i, pl.ds(j * 16, 16)] = ones16
    return carry
  lax.fori_loop(0, CHUNK, fillo, 0)

  def cnt_phase(ph, carry):
    pltpu.sync_copy(dst_hbm.at[pl.ds(row0 + ph * PHROWS, PHROWS)], idx_d)
    handles = []
    for k in range(PHROWS):
      handles.append(pltpu.async_copy(
          rows[k % 2], acc.at[idx_d.at[k]], ssems[k % 2], add=True))
    for h in handles:
      h.wait()
    return carry
  lax.fori_loop(0, NPH, cnt_phase, 0)

  plsc.subcore_barrier()

  for b in range(NZF):
    pltpu.sync_copy(acc.at[pl.ds(base + b * CHUNK, CHUNK)], rows_v)
    pltpu.sync_copy(rows_v, pcnt_hbm.at[pl.ds(hb + b * CHUNK, CHUNK)])
  pltpu.sync_copy(acc.at[pl.ds(base + NZF * CHUNK, RZ)], rows_v.at[pl.ds(0, RZ)])
  pltpu.sync_copy(rows_v.at[pl.ds(0, RZ)], pcnt_hbm.at[pl.ds(hb + NZF * CHUNK, RZ)])


@jax.jit
def _sc_aggregate(x, src2d, dst2d):
  mesh = plsc.VectorSubcoreMesh(core_axis_name="c", subcore_axis_name="s")
  return pl.kernel(
      _sc_body,
      mesh=mesh,
      out_type=(
          jax.ShapeDtypeStruct((NC * NP, D), jnp.float32),
          jax.ShapeDtypeStruct((NC * NP, D), jnp.float32),
      ),
      scratch_types=[
          pltpu.VMEM_SHARED((NP, D), jnp.float32),
          pltpu.VMEM((PHROWS, CHUNK), jnp.int32),
          pltpu.VMEM((PHROWS, CHUNK), jnp.int32),
          pltpu.VMEM((CHUNK, D), jnp.float32),
          pltpu.VMEM((CHUNK, D), jnp.float32),
          pltpu.SemaphoreType.DMA,
          pltpu.SemaphoreType.DMA,
          pltpu.SemaphoreType.DMA,
          pltpu.SemaphoreType.DMA,
      ],
  )(x, src2d, dst2d)


def _combine_body(ps_ref, pc_ref, out_ref):
  ssum = ps_ref[0:N, :] + ps_ref[NP:NP + N, :]
  cn = pc_ref[0:N, 0:1] + pc_ref[NP:NP + N, 0:1]
  out_ref[...] = ssum / jnp.maximum(cn, 1.0)


@jax.jit
def _tc_combine(psum, pcnt):
  return pl.pallas_call(
      _combine_body,
      out_shape=jax.ShapeDtypeStruct((N, D), jnp.float32),
  )(psum, pcnt)


def kernel(x, edge_index):
  src = edge_index[0].astype(jnp.int32)
  dst = edge_index[1].astype(jnp.int32)
  pad = EROWS * CHUNK - E
  src2d = jnp.concatenate([src, jnp.zeros((pad,), jnp.int32)]).reshape(EROWS, CHUNK)
  dst2d = jnp.concatenate([dst, jnp.full((pad,), NP - 1, jnp.int32)]).reshape(EROWS, CHUNK)
  psum, pcnt = _sc_aggregate(x, src2d, dst2d)
  return _tc_combine(psum, pcnt)


# speedup vs baseline: 1.8065x; 1.7369x over previous
import jax
import jax.numpy as jnp
from jax import lax
from jax.experimental import pallas as pl
from jax.experimental.pallas import tpu as pltpu
from jax.experimental.pallas import tpu_sc as plsc

N = 10000
NP = 10112
E = 320000
D = 128

NC = 2
NS = 16
NW = NC * NS
EPT = E // NW
CHUNK = 128
NFULL = EPT // CHUNK
TAIL = EPT - NFULL * CHUNK
RPT = NP // NS
NZF = RPT // CHUNK
RZ = RPT - NZF * CHUNK


def _sc_body(x_hbm, src_hbm, dst_hbm, psum_hbm, pcnt_hbm,
             acc, src_ring, dst_ring, rows_v, rows_b, src_t, dst_t, rows_t,
             sem, sem2):
  c = lax.axis_index("c")
  s = lax.axis_index("s")
  wid = s * NC + c

  zeros16 = jnp.zeros((16,), jnp.float32)
  ones16 = jnp.ones((16,), jnp.float32)

  def fillz(i, carry):
    for j in range(D // 16):
      rows_v[i, pl.ds(j * 16, 16)] = zeros16
    return carry
  lax.fori_loop(0, CHUNK, fillz, 0)

  base = s * RPT
  for b in range(NZF):
    pltpu.sync_copy(rows_v, acc.at[pl.ds(base + b * CHUNK, CHUNK)])
  pltpu.sync_copy(rows_v.at[pl.ds(0, RZ)], acc.at[pl.ds(base + NZF * CHUNK, RZ)])

  plsc.subcore_barrier()

  ebase = wid * EPT

  def edge_pair(i, carry):
    off0 = ebase + (2 * i) * CHUNK
    off1 = off0 + CHUNK
    pltpu.sync_copy(src_hbm.at[pl.ds(off0, CHUNK)], src_ring.at[0])
    pltpu.sync_copy(dst_hbm.at[pl.ds(off0, CHUNK)], dst_ring.at[0])
    pltpu.sync_copy(src_hbm.at[pl.ds(off1, CHUNK)], src_ring.at[1])
    pltpu.sync_copy(dst_hbm.at[pl.ds(off1, CHUNK)], dst_ring.at[1])
    ga = pltpu.async_copy(x_hbm.at[src_ring.at[0]], rows_v, sem)
    gb = pltpu.async_copy(x_hbm.at[src_ring.at[1]], rows_b, sem2)
    ga.wait()
    pltpu.sync_copy(rows_v, acc.at[dst_ring.at[0]], add=True)
    gb.wait()
    pltpu.sync_copy(rows_b, acc.at[dst_ring.at[1]], add=True)
    return carry
  lax.fori_loop(0, NFULL // 2, edge_pair, 0)

  toff = ebase + NFULL * CHUNK
  pltpu.sync_copy(src_hbm.at[pl.ds(toff, TAIL)], src_t)
  pltpu.sync_copy(dst_hbm.at[pl.ds(toff, TAIL)], dst_t)
  pltpu.async_copy(x_hbm.at[src_t], rows_t, sem).wait()
  pltpu.sync_copy(rows_t, acc.at[dst_t], add=True)

  plsc.subcore_barrier()

  hb = c * NP + base
  for b in range(NZF):
    pltpu.sync_copy(acc.at[pl.ds(base + b * CHUNK, CHUNK)], rows_v)
    pltpu.sync_copy(rows_v, psum_hbm.at[pl.ds(hb + b * CHUNK, CHUNK)])
  pltpu.sync_copy(acc.at[pl.ds(base + NZF * CHUNK, RZ)], rows_v.at[pl.ds(0, RZ)])
  pltpu.sync_copy(rows_v.at[pl.ds(0, RZ)], psum_hbm.at[pl.ds(hb + NZF * CHUNK, RZ)])

  lax.fori_loop(0, CHUNK, fillz, 0)
  for b in range(NZF):
    pltpu.sync_copy(rows_v, acc.at[pl.ds(base + b * CHUNK, CHUNK)])
  pltpu.sync_copy(rows_v.at[pl.ds(0, RZ)], acc.at[pl.ds(base + NZF * CHUNK, RZ)])

  plsc.subcore_barrier()

  def fillo(i, carry):
    for j in range(D // 16):
      rows_v[i, pl.ds(j * 16, 16)] = ones16
      rows_b[i, pl.ds(j * 16, 16)] = ones16
    return carry
  lax.fori_loop(0, CHUNK, fillo, 0)

  def cnt_pair(i, carry):
    off0 = ebase + (2 * i) * CHUNK
    off1 = off0 + CHUNK
    pltpu.sync_copy(dst_hbm.at[pl.ds(off0, CHUNK)], dst_ring.at[0])
    pltpu.sync_copy(dst_hbm.at[pl.ds(off1, CHUNK)], dst_ring.at[1])
    s0 = pltpu.async_copy(rows_v, acc.at[dst_ring.at[0]], sem, add=True)
    s1 = pltpu.async_copy(rows_b, acc.at[dst_ring.at[1]], sem2, add=True)
    s0.wait()
    s1.wait()
    return carry
  lax.fori_loop(0, NFULL // 2, cnt_pair, 0)

  pltpu.sync_copy(dst_hbm.at[pl.ds(toff, TAIL)], dst_t)
  pltpu.sync_copy(rows_v.at[pl.ds(0, TAIL)], acc.at[dst_t], add=True)

  plsc.subcore_barrier()

  for b in range(NZF):
    pltpu.sync_copy(acc.at[pl.ds(base + b * CHUNK, CHUNK)], rows_v)
    pltpu.sync_copy(rows_v, pcnt_hbm.at[pl.ds(hb + b * CHUNK, CHUNK)])
  pltpu.sync_copy(acc.at[pl.ds(base + NZF * CHUNK, RZ)], rows_v.at[pl.ds(0, RZ)])
  pltpu.sync_copy(rows_v.at[pl.ds(0, RZ)], pcnt_hbm.at[pl.ds(hb + NZF * CHUNK, RZ)])


@jax.jit
def _sc_aggregate(x, src, dst):
  mesh = plsc.VectorSubcoreMesh(core_axis_name="c", subcore_axis_name="s")
  return pl.kernel(
      _sc_body,
      mesh=mesh,
      out_type=(
          jax.ShapeDtypeStruct((NC * NP, D), jnp.float32),
          jax.ShapeDtypeStruct((NC * NP, D), jnp.float32),
      ),
      scratch_types=[
          pltpu.VMEM_SHARED((NP, D), jnp.float32),
          pltpu.VMEM((2, CHUNK), jnp.int32),
          pltpu.VMEM((2, CHUNK), jnp.int32),
          pltpu.VMEM((CHUNK, D), jnp.float32),
          pltpu.VMEM((CHUNK, D), jnp.float32),
          pltpu.VMEM((TAIL,), jnp.int32),
          pltpu.VMEM((TAIL,), jnp.int32),
          pltpu.VMEM((TAIL, D), jnp.float32),
          pltpu.SemaphoreType.DMA,
          pltpu.SemaphoreType.DMA,
      ],
  )(x, src, dst)


def _combine_body(ps_ref, pc_ref, out_ref):
  ssum = ps_ref[0:N, :] + ps_ref[NP:NP + N, :]
  cn = pc_ref[0:N, 0:1] + pc_ref[NP:NP + N, 0:1]
  out_ref[...] = ssum / jnp.maximum(cn, 1.0)


@jax.jit
def _tc_combine(psum, pcnt):
  return pl.pallas_call(
      _combine_body,
      out_shape=jax.ShapeDtypeStruct((N, D), jnp.float32),
  )(psum, pcnt)


def kernel(x, edge_index):
  src = edge_index[0].astype(jnp.int32)
  dst = edge_index[1].astype(jnp.int32)
  psum, pcnt = _sc_aggregate(x, src, dst)
  return _tc_combine(psum, pcnt)
